# Initial kernel scaffold; baseline (speedup 1.0000x reference)
import jax
import jax.numpy as jnp
from jax.experimental import pallas as pl


def kernel(hidden_states, U, W, a, b, gamma, beta, targets):
    raise NotImplementedError("write your pallas kernel here")


# speedup vs baseline: 503.5456x; 503.5456x over previous
import functools

import jax
import jax.numpy as jnp
from jax.experimental import pallas as pl

EPS = 1e-5


def _mm(a, b):
    return jax.lax.dot_general(
        a, b, (((1,), (0,)), ((), ())),
        preferred_element_type=jnp.float32,
        precision=jax.lax.Precision.HIGHEST,
    )


def _ln(x, gamma, beta):
    m = jnp.mean(x, axis=-1, keepdims=True)
    v = jnp.mean((x - m) ** 2, axis=-1, keepdims=True)
    return (x - m) * jax.lax.rsqrt(v + EPS) * gamma + beta


def _fast_weight_kernel(h_ref, u_ref, w_ref, a_ref, b_ref, g_ref, be_ref,
                        tgt_ref, out_ref):
    h = h_ref[:]
    gamma = g_ref[:]
    beta = be_ref[:]
    T = h.shape[0]

    z = jnp.maximum(_mm(h, u_ref[:]) + a_ref[:], 0.0)
    y = _mm(z, w_ref[:]) + b_ref[:]

    mu = jnp.mean(y, axis=-1, keepdims=True)
    var = jnp.mean((y - mu) ** 2, axis=-1, keepdims=True)
    rstd = jax.lax.rsqrt(var + EPS)
    xhat = (y - mu) * rstd
    yln = xhat * gamma + beta

    ymax = jnp.max(yln, axis=-1, keepdims=True)
    ey = jnp.exp(yln - ymax)
    p = ey / jnp.sum(ey, axis=-1, keepdims=True)
    qidx = jax.lax.broadcasted_iota(jnp.int32, yln.shape, 1)
    onehot = (qidx == tgt_ref[:]).astype(jnp.float32)
    dy = p - onehot

    dxh = dy * gamma
    g = rstd * (dxh
                - jnp.mean(dxh, axis=-1, keepdims=True)
                - xhat * jnp.mean(dxh * xhat, axis=-1, keepdims=True))

    row = jax.lax.broadcasted_iota(jnp.int32, (T, T), 0)
    col = jax.lax.broadcasted_iota(jnp.int32, (T, T), 1)
    strict = (col < row).astype(jnp.float32)
    incl = (col <= row).astype(jnp.float32)

    C = _mm(incl, h)
    u = h * z
    Mp = _mm(z, u.T) * strict
    S = C * _mm(Mp, g) - _mm(Mp, g * C)

    c = jnp.sum(h, axis=-1, keepdims=True)
    Gc = c * g
    Bsum = C * _mm(strict, Gc) - _mm(strict, Gc * C)

    out_ref[:] = _ln(y - S - Bsum, gamma, beta)


@functools.partial(jax.jit, static_argnames=("interpret",))
def kernel(hidden_states, U, W, a, b, gamma, beta, targets, interpret=False):
    h = hidden_states[0]
    T, H = h.shape
    out = pl.pallas_call(
        _fast_weight_kernel,
        out_shape=jax.ShapeDtypeStruct((T, H), jnp.float32),
        interpret=interpret,
    )(h.astype(jnp.float32),
      U.astype(jnp.float32),
      W.astype(jnp.float32),
      a.reshape(1, H).astype(jnp.float32),
      b.reshape(1, H).astype(jnp.float32),
      gamma.reshape(1, H).astype(jnp.float32),
      beta.reshape(1, H).astype(jnp.float32),
      targets.reshape(T, 1).astype(jnp.int32))
    return out[None]


# speedup vs baseline: 875.4968x; 1.7387x over previous
import functools

import jax
import jax.numpy as jnp
from jax.experimental import pallas as pl

EPS = 1e-5


def _mm(a, b):
    return jax.lax.dot_general(
        a, b, (((1,), (0,)), ((), ())),
        preferred_element_type=jnp.float32,
    )


def _ln(x, gamma, beta):
    m = jnp.mean(x, axis=-1, keepdims=True)
    v = jnp.mean((x - m) ** 2, axis=-1, keepdims=True)
    return (x - m) * jax.lax.rsqrt(v + EPS) * gamma + beta


def _fast_weight_kernel(h_ref, u_ref, w_ref, a_ref, b_ref, g_ref, be_ref,
                        tgt_ref, out_ref):
    h = h_ref[:]
    gamma = g_ref[:]
    beta = be_ref[:]
    T = h.shape[0]

    z = jnp.maximum(_mm(h, u_ref[:]) + a_ref[:], 0.0)
    y = _mm(z, w_ref[:]) + b_ref[:]

    mu = jnp.mean(y, axis=-1, keepdims=True)
    var = jnp.mean((y - mu) ** 2, axis=-1, keepdims=True)
    rstd = jax.lax.rsqrt(var + EPS)
    xhat = (y - mu) * rstd
    yln = xhat * gamma + beta

    ymax = jnp.max(yln, axis=-1, keepdims=True)
    ey = jnp.exp(yln - ymax)
    p = ey / jnp.sum(ey, axis=-1, keepdims=True)
    qidx = jax.lax.broadcasted_iota(jnp.int32, yln.shape, 1)
    onehot = (qidx == tgt_ref[:]).astype(jnp.float32)
    dy = p - onehot

    dxh = dy * gamma
    g = rstd * (dxh
                - jnp.mean(dxh, axis=-1, keepdims=True)
                - xhat * jnp.mean(dxh * xhat, axis=-1, keepdims=True))

    row = jax.lax.broadcasted_iota(jnp.int32, (T, T), 0)
    col = jax.lax.broadcasted_iota(jnp.int32, (T, T), 1)
    strict = (col < row).astype(jnp.float32)
    incl = (col <= row).astype(jnp.float32)

    C = _mm(incl, h)
    u = h * z
    Mp = _mm(z, u.T) * strict
    S = C * _mm(Mp, g) - _mm(Mp, g * C)

    c = jnp.sum(h, axis=-1, keepdims=True)
    Gc = c * g
    Bsum = C * _mm(strict, Gc) - _mm(strict, Gc * C)

    out_ref[:] = _ln(y - S - Bsum, gamma, beta)


@functools.partial(jax.jit, static_argnames=("interpret",))
def kernel(hidden_states, U, W, a, b, gamma, beta, targets, interpret=False):
    h = hidden_states[0]
    T, H = h.shape
    out = pl.pallas_call(
        _fast_weight_kernel,
        out_shape=jax.ShapeDtypeStruct((T, H), jnp.float32),
        interpret=interpret,
    )(h.astype(jnp.float32),
      U.astype(jnp.float32),
      W.astype(jnp.float32),
      a.reshape(1, H).astype(jnp.float32),
      b.reshape(1, H).astype(jnp.float32),
      gamma.reshape(1, H).astype(jnp.float32),
      beta.reshape(1, H).astype(jnp.float32),
      targets.reshape(T, 1).astype(jnp.int32))
    return out[None]
